# baseline (speedup 1.0000x reference)
import functools

import jax
import jax.numpy as jnp
from jax.experimental import pallas as pl
from jax.experimental.pallas import tpu as pltpu
from jax.experimental.pallas import tpu_sc as plsc

_NUM_EMB = 1024
_DIM = 16384
_BATCH = 8
_BK = 128


def _dist_body(flat_ref, w_ref, idx_ref, loss_ref, minval_ref, minidx_ref):
    k = pl.program_id(0)
    nk = pl.num_programs(0)
    flat = flat_ref[...]
    w = w_ref[...]
    dot = jax.lax.dot_general(
        flat, w, (((1,), (1,)), ((), ())),
        preferred_element_type=jnp.float32)
    w_sq = jnp.sum(w * w, axis=1)
    d2p = w_sq[None, :] - 2.0 * dot
    local_min = jnp.min(d2p, axis=1, keepdims=True)
    lane = jax.lax.broadcasted_iota(jnp.int32, d2p.shape, 1)
    local_arg = jnp.min(
        jnp.where(d2p == local_min, lane, _NUM_EMB), axis=1, keepdims=True
    ) + k * _BK

    @pl.when(k == 0)
    def _():
        minval_ref[...] = local_min
        minidx_ref[...] = local_arg

    @pl.when(k > 0)
    def _():
        better = local_min < minval_ref[...]
        minval_ref[...] = jnp.where(better, local_min, minval_ref[...])
        minidx_ref[...] = jnp.where(better, local_arg, minidx_ref[...])

    @pl.when(k == nk - 1)
    def _():
        x_sq = jnp.sum(flat * flat, axis=1, keepdims=True)
        d2min = minval_ref[...] + x_sq
        loss_ref[...] = (1.25 / (_BATCH * _DIM)) * jnp.sum(
            d2min, keepdims=True)
        idx_ref[...] = minidx_ref[...]


def _distances_argmin(flat, emb_weight):
    grid = _NUM_EMB // _BK
    idx, loss = pl.pallas_call(
        _dist_body,
        grid=(grid,),
        in_specs=[
            pl.BlockSpec((_BATCH, _DIM), lambda k: (0, 0)),
            pl.BlockSpec((_BK, _DIM), lambda k: (k, 0)),
        ],
        out_specs=[
            pl.BlockSpec((_BATCH, 1), lambda k: (0, 0)),
            pl.BlockSpec((1, 1), lambda k: (0, 0)),
        ],
        out_shape=[
            jax.ShapeDtypeStruct((_BATCH, 1), jnp.int32),
            jax.ShapeDtypeStruct((1, 1), jnp.float32),
        ],
        scratch_shapes=[
            pltpu.VMEM((_BATCH, 1), jnp.float32),
            pltpu.VMEM((_BATCH, 1), jnp.int32),
        ],
    )(flat, emb_weight)
    return idx, loss


def _sc_gather(emb_weight, idx):
    rows_per_core = _BATCH // 2

    @functools.partial(
        pl.kernel,
        out_type=jax.ShapeDtypeStruct((_BATCH, _DIM), jnp.float32),
        mesh=plsc.ScalarSubcoreMesh(axis_name="core", num_cores=2),
        scratch_types=[
            pltpu.SMEM((_BATCH,), jnp.int32),
            pltpu.SemaphoreType.DMA,
            pltpu.SemaphoreType.DMA,
        ],
    )
    def gather_kernel(idx_hbm, w_hbm, out_hbm, idx_smem, sem_idx, sem_rows):
        core = jax.lax.axis_index("core")
        pltpu.async_copy(idx_hbm, idx_smem, sem_idx).wait()

        @pl.loop(0, rows_per_core)
        def _(i):
            b = core * rows_per_core + i
            pltpu.async_copy(
                w_hbm.at[idx_smem[b]], out_hbm.at[b], sem_rows).wait()

    return gather_kernel(idx, emb_weight)


def kernel(inputs, emb_weight):
    B = inputs.shape[0]
    flat = inputs.reshape(B, -1)
    idx, loss = _distances_argmin(flat, emb_weight)
    quantized = _sc_gather(emb_weight, idx.reshape(B))
    return (
        quantized.reshape(inputs.shape),
        loss.reshape(()),
        idx,
    )
